# baseline (speedup 1.0000x reference)
import functools

import jax
import jax.numpy as jnp
from jax import lax
from jax.experimental import pallas as pl
from jax.experimental.pallas import tpu as pltpu
from jax.experimental.pallas import tpu_sc as plsc

N_NODES = 50000
N_EDGES = 800000
NODE_DIM = 64
EDGE_DIM = 32
HALF = N_EDGES // 2

NW = 32
EDGES_PER_W = N_EDGES // NW
OP_ROWS = 100
K_OPS = 5
CHUNK = OP_ROWS * K_OPS
N_CHUNKS = EDGES_PER_W // CHUNK
IDX_ROWS = EDGES_PER_W // OP_ROWS

_mesh = plsc.VectorSubcoreMesh(core_axis_name="c", subcore_axis_name="s")


@functools.partial(
    pl.kernel,
    mesh=_mesh,
    compiler_params=pltpu.CompilerParams(use_tc_tiling_on_sc=False),
    out_type=jax.ShapeDtypeStruct((HALF, 4 * EDGE_DIM), jnp.float32),
    scratch_types=[
        pltpu.VMEM((IDX_ROWS, OP_ROWS), jnp.int32),
        pltpu.VMEM((IDX_ROWS, OP_ROWS), jnp.int32),
        pltpu.VMEM((CHUNK, EDGE_DIM), jnp.float32),
        pltpu.VMEM((CHUNK, EDGE_DIM), jnp.float32),
        pltpu.VMEM((CHUNK, EDGE_DIM), jnp.float32),
        pltpu.VMEM((CHUNK, EDGE_DIM), jnp.float32),
        pltpu.SemaphoreType.DMA,
        pltpu.SemaphoreType.DMA,
        pltpu.SemaphoreType.DMA,
        pltpu.SemaphoreType.DMA,
    ],
)
def _gather_edges(ptab_hbm, qtab_hbm, ports_hbm, protos_hbm, out_hbm,
                  idxp_v, idxq_v, bufp0, bufq0, bufp1, bufq1,
                  sem_g0, sem_g1, sem_w0, sem_w1):
    wid = lax.axis_index("s") * 2 + lax.axis_index("c")
    hi = wid >= 16
    row0 = (wid % 16) * EDGES_PER_W
    bufp = (bufp0, bufp1)
    bufq = (bufq0, bufq1)
    sem_g = (sem_g0, sem_g1)
    sem_w = (sem_w0, sem_w1)

    idx_base = pl.multiple_of(wid * IDX_ROWS, 2)
    pltpu.sync_copy(ports_hbm.at[pl.ds(idx_base, IDX_ROWS)], idxp_v)
    pltpu.sync_copy(protos_hbm.at[pl.ds(idx_base, IDX_ROWS)], idxq_v)

    def fire_gathers(c, slot):
        for j in range(K_OPS):
            sl = pl.ds(j * OP_ROWS, OP_ROWS)
            pltpu.async_copy(ptab_hbm.at[idxp_v.at[c * K_OPS + j]],
                             bufp[slot].at[sl], sem_g[slot])
            pltpu.async_copy(qtab_hbm.at[idxq_v.at[c * K_OPS + j]],
                             bufq[slot].at[sl], sem_g[slot])

    def drain_gathers(slot):
        dummy = ptab_hbm.at[pl.ds(0, CHUNK)]
        pltpu.make_async_copy(dummy, bufp[slot], sem_g[slot]).wait()
        pltpu.make_async_copy(dummy, bufq[slot], sem_g[slot]).wait()

    def fire_writes(c, slot):
        r = pl.multiple_of(row0 + c * CHUNK, CHUNK)
        rows = out_hbm.at[pl.ds(r, CHUNK)]

        @pl.when(hi)
        def _():
            pltpu.async_copy(bufp[slot],
                             rows.at[:, pl.ds(2 * EDGE_DIM, EDGE_DIM)],
                             sem_w[slot])
            pltpu.async_copy(bufq[slot],
                             rows.at[:, pl.ds(3 * EDGE_DIM, EDGE_DIM)],
                             sem_w[slot])

        @pl.when(jnp.logical_not(hi))
        def _():
            pltpu.async_copy(bufp[slot],
                             rows.at[:, pl.ds(0, EDGE_DIM)], sem_w[slot])
            pltpu.async_copy(bufq[slot],
                             rows.at[:, pl.ds(EDGE_DIM, EDGE_DIM)],
                             sem_w[slot])

    def wait_writes(slot):
        rows = out_hbm.at[pl.ds(0, CHUNK)]
        pltpu.make_async_copy(bufp[slot], rows.at[:, pl.ds(0, EDGE_DIM)],
                              sem_w[slot]).wait()
        pltpu.make_async_copy(bufq[slot], rows.at[:, pl.ds(EDGE_DIM,
                                                           EDGE_DIM)],
                              sem_w[slot]).wait()

    fire_gathers(0, 0)
    drain_gathers(0)
    fire_writes(0, 0)
    fire_gathers(1, 1)

    def body(k, carry):
        c0 = 2 * k
        c1 = 2 * k + 1
        drain_gathers(1)
        fire_writes(c1 - 2, 1)
        wait_writes(0)
        fire_gathers(c0, 0)
        drain_gathers(0)
        fire_writes(c0, 0)
        wait_writes(1)
        fire_gathers(c1, 1)
        return carry

    lax.fori_loop(1, N_CHUNKS // 2, body, 0)
    drain_gathers(1)
    fire_writes(N_CHUNKS - 1, 1)
    wait_writes(0)
    wait_writes(1)


def _transpose_body(x_ref, o_ref):
    h = pl.program_id(1)
    x = x_ref[...]
    half = jnp.where(h == 0, x[:, :NODE_DIM], x[:, NODE_DIM:])
    o_ref[...] = half.T


_T_ROWS = 640
_T_GRID = HALF // _T_ROWS


def _retile(g):
    return pl.pallas_call(
        _transpose_body,
        grid=(_T_GRID, 2),
        in_specs=[pl.BlockSpec((_T_ROWS, 2 * NODE_DIM),
                               lambda i, h: (i, 0))],
        out_specs=pl.BlockSpec((NODE_DIM, _T_ROWS),
                               lambda i, h: (0, h * _T_GRID + i)),
        out_shape=jax.ShapeDtypeStruct((NODE_DIM, N_EDGES), jnp.float32),
    )(g)


def _mm_body(x_ref, w_ref, b_ref, o_ref):
    o_ref[...] = (
        lax.dot_general(
            x_ref[...], w_ref[...],
            (((1,), (1,)), ((), ())),
            preferred_element_type=jnp.float32,
        )
        + b_ref[...]
    )


_MM_BLOCK = 2000


def _node_proj(ip_bits, W_ip, b_ip):
    return pl.pallas_call(
        _mm_body,
        grid=(N_NODES // _MM_BLOCK,),
        in_specs=[
            pl.BlockSpec((_MM_BLOCK, 32), lambda i: (i, 0)),
            pl.BlockSpec((NODE_DIM, 32), lambda i: (0, 0)),
            pl.BlockSpec((1, NODE_DIM), lambda i: (0, 0)),
        ],
        out_specs=pl.BlockSpec((_MM_BLOCK, NODE_DIM), lambda i: (i, 0)),
        out_shape=jax.ShapeDtypeStruct((N_NODES, NODE_DIM), jnp.float32),
    )(ip_bits, W_ip, b_ip.reshape(1, NODE_DIM))


def kernel(ip_bits, ports, protos, W_ip, b_ip, port_table, proto_table):
    ports2 = ports.astype(jnp.int32).reshape(N_EDGES // OP_ROWS, OP_ROWS)
    protos2 = protos.astype(jnp.int32).reshape(N_EDGES // OP_ROWS, OP_ROWS)
    g = _gather_edges(port_table, proto_table, ports2, protos2)
    edge_attr = _retile(g).T
    x_embedded = _node_proj(ip_bits, W_ip, b_ip)
    return (x_embedded, edge_attr)


# speedup vs baseline: 1.6164x; 1.6164x over previous
import functools

import jax
import jax.numpy as jnp
from jax import lax
from jax.experimental import pallas as pl
from jax.experimental.pallas import tpu as pltpu
from jax.experimental.pallas import tpu_sc as plsc

N_NODES = 50000
N_EDGES = 800000
NODE_DIM = 64
EDGE_DIM = 32
HALF = N_EDGES // 2

NW = 32
EDGES_PER_W = N_EDGES // NW
OP_ROWS = 125
K_OPS = 4
CHUNK = OP_ROWS * K_OPS
N_CHUNKS = EDGES_PER_W // CHUNK
IDX_ROWS = EDGES_PER_W // OP_ROWS

_mesh = plsc.VectorSubcoreMesh(core_axis_name="c", subcore_axis_name="s")


@functools.partial(
    pl.kernel,
    mesh=_mesh,
    compiler_params=pltpu.CompilerParams(use_tc_tiling_on_sc=False),
    out_type=jax.ShapeDtypeStruct((HALF, 4 * EDGE_DIM), jnp.float32),
    scratch_types=[
        pltpu.VMEM((IDX_ROWS, OP_ROWS), jnp.int32),
        pltpu.VMEM((IDX_ROWS, OP_ROWS), jnp.int32),
        pltpu.VMEM((CHUNK, EDGE_DIM), jnp.float32),
        pltpu.VMEM((CHUNK, EDGE_DIM), jnp.float32),
        pltpu.VMEM((CHUNK, EDGE_DIM), jnp.float32),
        pltpu.VMEM((CHUNK, EDGE_DIM), jnp.float32),
        pltpu.SemaphoreType.DMA,
        pltpu.SemaphoreType.DMA,
        pltpu.SemaphoreType.DMA,
        pltpu.SemaphoreType.DMA,
    ],
)
def _gather_edges(ptab_hbm, qtab_hbm, ports_hbm, protos_hbm, out_hbm,
                  idxp_v, idxq_v, bufp0, bufq0, bufp1, bufq1,
                  sem_g0, sem_g1, sem_w0, sem_w1):
    wid = lax.axis_index("s") * 2 + lax.axis_index("c")
    col0 = pl.multiple_of((wid // 16) * 2 * EDGE_DIM, 2 * EDGE_DIM)
    row0 = (wid % 16) * EDGES_PER_W
    bufp = (bufp0, bufp1)
    bufq = (bufq0, bufq1)
    sem_g = (sem_g0, sem_g1)
    sem_w = (sem_w0, sem_w1)

    idx_base = pl.multiple_of(wid * IDX_ROWS, 8)
    pltpu.sync_copy(ports_hbm.at[pl.ds(idx_base, IDX_ROWS)], idxp_v)
    pltpu.sync_copy(protos_hbm.at[pl.ds(idx_base, IDX_ROWS)], idxq_v)

    def fire_gathers(c, slot):
        for j in range(K_OPS):
            sl = pl.ds(j * OP_ROWS, OP_ROWS)
            pltpu.async_copy(ptab_hbm.at[idxp_v.at[c * K_OPS + j]],
                             bufp[slot].at[sl], sem_g[slot])
            pltpu.async_copy(qtab_hbm.at[idxq_v.at[c * K_OPS + j]],
                             bufq[slot].at[sl], sem_g[slot])

    def drain_gathers(slot):
        dummy = ptab_hbm.at[pl.ds(0, CHUNK)]
        pltpu.make_async_copy(dummy, bufp[slot], sem_g[slot]).wait()
        pltpu.make_async_copy(dummy, bufq[slot], sem_g[slot]).wait()

    def fire_writes(c, slot):
        r = pl.multiple_of(row0 + c * CHUNK, CHUNK)
        rows = out_hbm.at[pl.ds(r, CHUNK)]
        pltpu.async_copy(bufp[slot], rows.at[:, pl.ds(col0, EDGE_DIM)],
                         sem_w[slot])
        pltpu.async_copy(bufq[slot],
                         rows.at[:, pl.ds(col0 + EDGE_DIM, EDGE_DIM)],
                         sem_w[slot])

    def wait_writes(slot):
        rows = out_hbm.at[pl.ds(0, CHUNK)]
        pltpu.make_async_copy(bufp[slot], rows.at[:, pl.ds(0, EDGE_DIM)],
                              sem_w[slot]).wait()
        pltpu.make_async_copy(bufq[slot], rows.at[:, pl.ds(EDGE_DIM,
                                                           EDGE_DIM)],
                              sem_w[slot]).wait()

    fire_gathers(0, 0)
    drain_gathers(0)
    fire_writes(0, 0)
    fire_gathers(1, 1)

    def body(k, carry):
        c0 = 2 * k
        c1 = 2 * k + 1
        drain_gathers(1)
        fire_writes(c1 - 2, 1)
        wait_writes(0)
        fire_gathers(c0, 0)
        drain_gathers(0)
        fire_writes(c0, 0)
        wait_writes(1)
        fire_gathers(c1, 1)
        return carry

    lax.fori_loop(1, N_CHUNKS // 2, body, 0)
    drain_gathers(1)
    fire_writes(N_CHUNKS - 1, 1)
    wait_writes(0)
    wait_writes(1)


def _transpose_body(x_ref, o_ref):
    h = pl.program_id(1)
    x = x_ref[...]
    half = jnp.where(h == 0, x[:, :NODE_DIM], x[:, NODE_DIM:])
    o_ref[...] = half.T


_T_ROWS = 640
_T_GRID = HALF // _T_ROWS


def _retile(g):
    return pl.pallas_call(
        _transpose_body,
        grid=(_T_GRID, 2),
        in_specs=[pl.BlockSpec((_T_ROWS, 2 * NODE_DIM),
                               lambda i, h: (i, 0))],
        out_specs=pl.BlockSpec((NODE_DIM, _T_ROWS),
                               lambda i, h: (0, h * _T_GRID + i)),
        out_shape=jax.ShapeDtypeStruct((NODE_DIM, N_EDGES), jnp.float32),
    )(g)


def _mm_body(x_ref, w_ref, b_ref, o_ref):
    o_ref[...] = (
        lax.dot_general(
            x_ref[...], w_ref[...],
            (((1,), (1,)), ((), ())),
            preferred_element_type=jnp.float32,
        )
        + b_ref[...]
    )


_MM_BLOCK = 2000


def _node_proj(ip_bits, W_ip, b_ip):
    return pl.pallas_call(
        _mm_body,
        grid=(N_NODES // _MM_BLOCK,),
        in_specs=[
            pl.BlockSpec((_MM_BLOCK, 32), lambda i: (i, 0)),
            pl.BlockSpec((NODE_DIM, 32), lambda i: (0, 0)),
            pl.BlockSpec((1, NODE_DIM), lambda i: (0, 0)),
        ],
        out_specs=pl.BlockSpec((_MM_BLOCK, NODE_DIM), lambda i: (i, 0)),
        out_shape=jax.ShapeDtypeStruct((N_NODES, NODE_DIM), jnp.float32),
    )(ip_bits, W_ip, b_ip.reshape(1, NODE_DIM))


def kernel(ip_bits, ports, protos, W_ip, b_ip, port_table, proto_table):
    ports2 = ports.astype(jnp.int32).reshape(N_EDGES // OP_ROWS, OP_ROWS)
    protos2 = protos.astype(jnp.int32).reshape(N_EDGES // OP_ROWS, OP_ROWS)
    g = _gather_edges(port_table, proto_table, ports2, protos2)
    edge_attr = (g.reshape(HALF, 2, NODE_DIM).transpose(2, 1, 0)
                 .reshape(NODE_DIM, N_EDGES).T)
    x_embedded = _node_proj(ip_bits, W_ip, b_ip)
    return (x_embedded, edge_attr)


# speedup vs baseline: 1.6321x; 1.0097x over previous
import functools

import jax
import jax.numpy as jnp
from jax import lax
from jax.experimental import pallas as pl
from jax.experimental.pallas import tpu as pltpu
from jax.experimental.pallas import tpu_sc as plsc

N_NODES = 50000
N_EDGES = 800000
NODE_DIM = 64
EDGE_DIM = 32

NW = 32
EDGES_PER_W = N_EDGES // NW
OP_ROWS = 125
K_OPS = 8
CHUNK = OP_ROWS * K_OPS
N_CHUNKS = EDGES_PER_W // CHUNK

_mesh = plsc.VectorSubcoreMesh(core_axis_name="c", subcore_axis_name="s")


@functools.partial(
    pl.kernel,
    mesh=_mesh,
    compiler_params=pltpu.CompilerParams(use_tc_tiling_on_sc=False),
    out_type=jax.ShapeDtypeStruct((N_EDGES, 2 * EDGE_DIM), jnp.float32),
    scratch_types=[
        pltpu.VMEM((K_OPS, OP_ROWS), jnp.int32),
        pltpu.VMEM((K_OPS, OP_ROWS), jnp.int32),
        pltpu.VMEM((CHUNK, EDGE_DIM), jnp.float32),
        pltpu.VMEM((CHUNK, EDGE_DIM), jnp.float32),
        pltpu.SemaphoreType.DMA,
    ],
)
def _gather_edges(ptab_hbm, qtab_hbm, ports_hbm, protos_hbm, out_hbm,
                  idxp_v, idxq_v, bufp_v, bufq_v, sem):
    wid = lax.axis_index("s") * 2 + lax.axis_index("c")
    base = wid * EDGES_PER_W

    def body(i, carry):
        off = pl.multiple_of(base + i * CHUNK, CHUNK)
        idx_off = pl.multiple_of(
            wid * (EDGES_PER_W // OP_ROWS) + i * K_OPS, K_OPS)
        pltpu.sync_copy(ports_hbm.at[pl.ds(idx_off, K_OPS)], idxp_v)
        pltpu.sync_copy(protos_hbm.at[pl.ds(idx_off, K_OPS)], idxq_v)
        handles = []
        for j in range(K_OPS):
            sl = pl.ds(j * OP_ROWS, OP_ROWS)
            handles.append(pltpu.async_copy(
                ptab_hbm.at[idxp_v.at[j]], bufp_v.at[sl], sem))
            handles.append(pltpu.async_copy(
                qtab_hbm.at[idxq_v.at[j]], bufq_v.at[sl], sem))
        for h in handles:
            h.wait()
        rows = out_hbm.at[pl.ds(off, CHUNK)]
        pltpu.sync_copy(bufp_v, rows.at[:, pl.ds(0, EDGE_DIM)])
        pltpu.sync_copy(bufq_v, rows.at[:, pl.ds(EDGE_DIM, EDGE_DIM)])
        return carry

    lax.fori_loop(0, N_CHUNKS, body, 0)


def _mm_body(x_ref, w_ref, b_ref, o_ref):
    o_ref[...] = (
        lax.dot_general(
            x_ref[...], w_ref[...],
            (((1,), (1,)), ((), ())),
            preferred_element_type=jnp.float32,
        )
        + b_ref[...]
    )


_MM_BLOCK = 2000


def _node_proj(ip_bits, W_ip, b_ip):
    return pl.pallas_call(
        _mm_body,
        grid=(N_NODES // _MM_BLOCK,),
        in_specs=[
            pl.BlockSpec((_MM_BLOCK, 32), lambda i: (i, 0)),
            pl.BlockSpec((NODE_DIM, 32), lambda i: (0, 0)),
            pl.BlockSpec((1, NODE_DIM), lambda i: (0, 0)),
        ],
        out_specs=pl.BlockSpec((_MM_BLOCK, NODE_DIM), lambda i: (i, 0)),
        out_shape=jax.ShapeDtypeStruct((N_NODES, NODE_DIM), jnp.float32),
    )(ip_bits, W_ip, b_ip.reshape(1, NODE_DIM))


def kernel(ip_bits, ports, protos, W_ip, b_ip, port_table, proto_table):
    ports2 = ports.astype(jnp.int32).reshape(N_EDGES // OP_ROWS, OP_ROWS)
    protos2 = protos.astype(jnp.int32).reshape(N_EDGES // OP_ROWS, OP_ROWS)
    edge_attr = _gather_edges(port_table, proto_table, ports2, protos2)
    x_embedded = _node_proj(ip_bits, W_ip, b_ip)
    return (x_embedded, edge_attr)
